# baseline (speedup 1.0000x reference)
import functools

import jax
import jax.numpy as jnp
from jax import lax
from jax.experimental import pallas as pl
from jax.experimental.pallas import tpu as pltpu
from jax.experimental.pallas import tpu_sc as plsc

N_NODES = 10000
D_IN = 256
D_LAT = 64

NC = 2
NS = 16
CHUNK = 128
NACC = 10240
RPT = NACC // NS
NACC_A = 10240
RPT_A = NACC_A // NS
DEG_W = 128
MB = 1000


def _deg_hist(dst32, ones_rows, zrows):
    nch = dst32.shape[1]
    mesh = plsc.VectorSubcoreMesh(core_axis_name="c", subcore_axis_name="s")

    @functools.partial(
        pl.kernel,
        out_type=jax.ShapeDtypeStruct((NC, NACC, DEG_W), jnp.float32),
        mesh=mesh,
        scratch_types=[
            pltpu.VMEM((nch, CHUNK), jnp.int32),
            pltpu.VMEM((CHUNK, DEG_W), jnp.float32),
            pltpu.VMEM_SHARED((NACC, DEG_W), jnp.float32),
            pltpu.SemaphoreType.DMA,
        ],
    )
    def deg_kernel(dst_h, ones_h, z_h, out_h, dst_v, rows_v, acc_sh, ssem):
        c = lax.axis_index("c")
        s = lax.axis_index("s")
        w = c * NS + s
        pltpu.sync_copy(dst_h.at[w], dst_v)
        pltpu.sync_copy(ones_h, rows_v)
        pltpu.sync_copy(z_h, acc_sh.at[pl.ds(s * RPT, RPT)])
        plsc.subcore_barrier()

        def swait():
            pltpu.make_async_copy(rows_v, acc_sh.at[dst_v.at[0]], ssem).wait()

        def step(j, carry):
            pltpu.async_copy(rows_v, acc_sh.at[dst_v.at[j]], ssem, add=True)

            @pl.when(j >= 3)
            def _():
                swait()

            return carry

        lax.fori_loop(0, nch, step, 0)
        swait()
        swait()
        swait()
        plsc.subcore_barrier()
        pltpu.sync_copy(acc_sh.at[pl.ds(s * RPT, RPT)],
                        out_h.at[c, pl.ds(s * RPT, RPT)])

    return deg_kernel(dst32, ones_rows, zrows)


def _make_agg(nch_tot, n_halves):
    mesh = plsc.VectorSubcoreMesh(core_axis_name="c", subcore_axis_name="s")
    nch = nch_tot // NC

    @functools.partial(
        pl.kernel,
        out_type=jax.ShapeDtypeStruct((n_halves, NC, NACC_A, 128), jnp.float32),
        mesh=mesh,
        scratch_types=[
            pltpu.VMEM((3, 4, 2, CHUNK), jnp.int32),
            pltpu.VMEM((2, CHUNK, 128), jnp.float32),
            pltpu.VMEM_SHARED((NACC_A, 128), jnp.float32),
            pltpu.SemaphoreType.DMA,
            pltpu.SemaphoreType.DMA,
            pltpu.SemaphoreType.DMA,
        ],
    )
    def agg_kernel(ta_h, sd_h, z_h, out_h, idxr, rows2, acc_sh,
                   gsem, isem, ssem):
        c = lax.axis_index("c")
        s = lax.axis_index("s")
        ng = nch // 4

        def gather(G, k, p):
            pltpu.async_copy(ta_h.at[idxr.at[G, k, 0]], rows2.at[p], gsem)

        def gwait():
            pltpu.make_async_copy(ta_h.at[idxr.at[0, 0, 0]], rows2.at[0],
                                  gsem).wait()

        def iwait():
            pltpu.make_async_copy(ta_h.at[idxr.at[0, 0, 0]], idxr.at[0],
                                  isem).wait()

        def swait():
            pltpu.make_async_copy(rows2.at[0], acc_sh.at[idxr.at[0, 0, 1]],
                                  ssem).wait()

        for h in range(n_halves):
            def ifetch(gg, G):
                pltpu.async_copy(sd_h.at[h, s, pl.ds(c * nch + gg * 4, 4)],
                                 idxr.at[G], isem)

            pltpu.sync_copy(z_h, acc_sh.at[pl.ds(s * RPT_A, RPT_A)])
            ifetch(0, 0)
            iwait()
            ifetch(jnp.minimum(1, ng - 1), 1)
            plsc.subcore_barrier()
            gather(0, 0, 0)

            def step(g, carry):
                G = lax.rem(g, 3)
                Gn = lax.rem(g + 1, 3)
                iwait()
                for k in range(4):
                    p = k % 2
                    if k == 0:
                        @pl.when(g >= 1)
                        def _():
                            swait()
                    else:
                        swait()
                    if k < 3:
                        gather(G, k + 1, 1 - p)
                    else:
                        gather(Gn, 0, 1 - p)
                    gwait()
                    pltpu.async_copy(rows2.at[p], acc_sh.at[idxr.at[G, k, 1]],
                                     ssem, add=True)
                    if k == 1:
                        ifetch(jnp.minimum(g + 2, ng - 1), lax.rem(g + 2, 3))
                return carry

            lax.fori_loop(0, ng, step, 0)
            iwait()
            gwait()
            swait()
            plsc.subcore_barrier()
            pltpu.sync_copy(acc_sh.at[pl.ds(s * RPT_A, RPT_A)],
                            out_h.at[h, c, pl.ds(s * RPT_A, RPT_A)])
            if h + 1 < n_halves:
                plsc.subcore_barrier()

    return agg_kernel


def _tc_matmul1(x, W1):
    def body(x_ref, w_ref, h1_ref):
        h1_ref[0] = jnp.dot(x_ref[:], w_ref[:],
                            preferred_element_type=jnp.float32)

    return pl.pallas_call(
        body,
        grid=(2, N_NODES // MB),
        in_specs=[
            pl.BlockSpec((MB, D_IN), lambda h, i: (i, 0)),
            pl.BlockSpec((D_IN, 128), lambda h, i: (0, h)),
        ],
        out_specs=pl.BlockSpec((1, MB, 128), lambda h, i: (h, i, 0)),
        out_shape=jax.ShapeDtypeStruct((2, N_NODES, 128), jnp.float32),
    )(x, W1)


def _tc_prescale(h1, degp):
    def body(h1_ref, degp_ref, hp_ref, dinv_ref):
        deg = degp_ref[0, :, :1] + degp_ref[1, :, :1] + 1.0
        dinv = lax.rsqrt(deg)
        hp_ref[0] = h1_ref[0] * dinv
        dinv_ref[:] = dinv

    return pl.pallas_call(
        body,
        grid=(2, N_NODES // MB),
        in_specs=[
            pl.BlockSpec((1, MB, 128), lambda h, i: (h, i, 0)),
            pl.BlockSpec((2, MB, DEG_W), lambda h, i: (0, i, 0)),
        ],
        out_specs=[
            pl.BlockSpec((1, MB, 128), lambda h, i: (h, i, 0)),
            pl.BlockSpec((MB, 1), lambda h, i: (i, 0)),
        ],
        out_shape=[
            jax.ShapeDtypeStruct((2, N_NODES, 128), jnp.float32),
            jax.ShapeDtypeStruct((N_NODES, 1), jnp.float32),
        ],
    )(h1, degp)


def _tc_bn_relu_matmul(agg1, hp1, dinv, b1s, gs, bes, Wc):
    def body(agg_ref, hp_ref, dinv_ref, b1_ref, g_ref, be_ref, wc_ref, out_ref):
        hid = pl.program_id(0)
        dinv = dinv_ref[:]
        h = (agg_ref[0, 0, :N_NODES, :] + agg_ref[0, 1, :N_NODES, :]
             + hp_ref[0]) * dinv + b1_ref[0, 0]
        mean = jnp.mean(h, axis=0, keepdims=True)
        var = jnp.mean(h * h, axis=0, keepdims=True) - mean * mean
        hn = (h - mean) * lax.rsqrt(var + 1e-5) * g_ref[0, 0] + be_ref[0, 0]
        hr = jnp.maximum(hn, 0.0)
        part = jnp.dot(hr, wc_ref[:], preferred_element_type=jnp.float32)

        @pl.when(hid == 0)
        def _():
            out_ref[:] = part

        @pl.when(hid == 1)
        def _():
            out_ref[:] = (out_ref[:] + part) * dinv

    return pl.pallas_call(
        body,
        grid=(2,),
        in_specs=[
            pl.BlockSpec((1, NC, NACC_A, 128), lambda h: (h, 0, 0, 0)),
            pl.BlockSpec((1, N_NODES, 128), lambda h: (h, 0, 0)),
            pl.BlockSpec((N_NODES, 1), lambda h: (0, 0)),
            pl.BlockSpec((1, 1, 128), lambda h: (h, 0, 0)),
            pl.BlockSpec((1, 1, 128), lambda h: (h, 0, 0)),
            pl.BlockSpec((1, 1, 128), lambda h: (h, 0, 0)),
            pl.BlockSpec((128, 128), lambda h: (h, 0)),
        ],
        out_specs=pl.BlockSpec((N_NODES, 128), lambda h: (0, 0)),
        out_shape=jax.ShapeDtypeStruct((N_NODES, 128), jnp.float32),
    )(agg1, hp1, dinv, b1s, gs, bes, Wc)


def _tc_final(agg2, gp, dinv, bc):
    def body(agg_ref, gp_ref, dinv_ref, b_ref, out_ref):
        v = ((agg_ref[0] + agg_ref[1] + gp_ref[:]) * dinv_ref[:] + b_ref[:])
        out_ref[0] = v[:, :D_LAT]
        out_ref[1] = v[:, D_LAT:]

    return pl.pallas_call(
        body,
        grid=(N_NODES // MB,),
        in_specs=[
            pl.BlockSpec((2, MB, 128), lambda i: (0, i, 0)),
            pl.BlockSpec((MB, 128), lambda i: (i, 0)),
            pl.BlockSpec((MB, 1), lambda i: (i, 0)),
            pl.BlockSpec((1, 128), lambda i: (0, 0)),
        ],
        out_specs=pl.BlockSpec((2, MB, D_LAT), lambda i: (0, i, 0)),
        out_shape=jax.ShapeDtypeStruct((2, N_NODES, D_LAT), jnp.float32),
    )(agg2, gp, dinv, bc)


def kernel(x, edge_index, W1, b1, gamma, beta, W_mu, b_mu, W_ls, b_ls):
    n_edges = edge_index.shape[1]
    e_pad = ((n_edges + NS * CHUNK - 1) // (NS * CHUNK)) * (NS * CHUNK)
    e_pad = ((e_pad + NC * NS * CHUNK - 1) // (NC * NS * CHUNK)) * (NC * NS * CHUNK)
    pad = e_pad - n_edges

    src = edge_index[0]
    dst = edge_index[1]
    ar = jnp.arange(pad, dtype=jnp.int32)
    srcp = jnp.concatenate([src, ar % N_NODES])
    dstp = jnp.concatenate([dst, N_NODES + ar % (NACC_A - N_NODES)])
    src16 = srcp.reshape(NS, -1, CHUNK)
    dst16 = dstp.reshape(NS, -1, CHUNK)
    sd16 = jnp.stack([src16, dst16], axis=2)
    dst32 = dstp.reshape(NC * NS, -1, CHUNK)

    ones_rows = jnp.ones((CHUNK, DEG_W), jnp.float32)
    z16 = jnp.zeros((RPT, DEG_W), jnp.float32)
    z128 = jnp.zeros((RPT_A, 128), jnp.float32)

    h1 = _tc_matmul1(x, W1)
    degp = _deg_hist(dst32, ones_rows, z16)

    hp1, dinv = _tc_prescale(h1, degp)

    sd2 = jnp.stack([sd16, sd16 + jnp.array([N_NODES, 0], jnp.int32)[None, None, :, None]])
    agg1 = _make_agg(src16.shape[1], 2)(hp1.reshape(2 * N_NODES, 128), sd2, z128)

    gp = _tc_bn_relu_matmul(
        agg1, hp1, dinv,
        b1.reshape(2, 1, 128), gamma.reshape(2, 1, 128), beta.reshape(2, 1, 128),
        jnp.concatenate([W_mu, W_ls], axis=1))

    agg2 = _make_agg(src16.shape[1], 1)(gp, sd16[None], z128)
    agg2 = agg2.reshape(NC, NACC_A, 128)

    out = _tc_final(agg2, gp, dinv,
                    jnp.concatenate([b_mu, b_ls]).reshape(1, 128))
    return (out[0], out[1])


# speedup vs baseline: 1.0028x; 1.0028x over previous
import functools

import jax
import jax.numpy as jnp
from jax import lax
from jax.experimental import pallas as pl
from jax.experimental.pallas import tpu as pltpu
from jax.experimental.pallas import tpu_sc as plsc

N_NODES = 10000
D_IN = 256
D_LAT = 64

NC = 2
NS = 16
CHUNK = 128
NACC = 10240
RPT = NACC // NS
NACC_A = 10240
RPT_A = NACC_A // NS
DEG_W = 128
MB = 1000


def _deg_hist(dst32, ones_rows, zrows):
    nch = dst32.shape[1]
    mesh = plsc.VectorSubcoreMesh(core_axis_name="c", subcore_axis_name="s")

    @functools.partial(
        pl.kernel,
        out_type=jax.ShapeDtypeStruct((NC, NACC, DEG_W), jnp.float32),
        mesh=mesh,
        scratch_types=[
            pltpu.VMEM((nch, CHUNK), jnp.int32),
            pltpu.VMEM((CHUNK, DEG_W), jnp.float32),
            pltpu.VMEM_SHARED((NACC, DEG_W), jnp.float32),
            pltpu.SemaphoreType.DMA,
        ],
    )
    def deg_kernel(dst_h, ones_h, z_h, out_h, dst_v, rows_v, acc_sh, ssem):
        c = lax.axis_index("c")
        s = lax.axis_index("s")
        w = c * NS + s
        pltpu.sync_copy(dst_h.at[w], dst_v)
        pltpu.sync_copy(ones_h, rows_v)
        pltpu.sync_copy(z_h, acc_sh.at[pl.ds(s * RPT, RPT)])
        plsc.subcore_barrier()

        def swait():
            pltpu.make_async_copy(rows_v, acc_sh.at[dst_v.at[0]], ssem).wait()

        def step(j, carry):
            pltpu.async_copy(rows_v, acc_sh.at[dst_v.at[j]], ssem, add=True)

            @pl.when(j >= 3)
            def _():
                swait()

            return carry

        lax.fori_loop(0, nch, step, 0)
        swait()
        swait()
        swait()
        plsc.subcore_barrier()
        pltpu.sync_copy(acc_sh.at[pl.ds(s * RPT, RPT)],
                        out_h.at[c, pl.ds(s * RPT, RPT)])

    return deg_kernel(dst32, ones_rows, zrows)


def _make_agg(nch_tot, n_halves):
    mesh = plsc.VectorSubcoreMesh(core_axis_name="c", subcore_axis_name="s")
    nch = nch_tot // NC

    @functools.partial(
        pl.kernel,
        out_type=jax.ShapeDtypeStruct((n_halves, NC, NACC_A, 128), jnp.float32),
        mesh=mesh,
        scratch_types=[
            pltpu.VMEM((4, 2, CHUNK), jnp.int32),
            pltpu.VMEM((2, CHUNK, 128), jnp.float32),
            pltpu.VMEM_SHARED((NACC_A, 128), jnp.float32),
            pltpu.SemaphoreType.DMA,
            pltpu.SemaphoreType.DMA,
            pltpu.SemaphoreType.DMA,
        ],
    )
    def agg_kernel(ta_h, sd_h, z_h, out_h, idxr, rows2, acc_sh,
                   gsem, isem, ssem):
        c = lax.axis_index("c")
        s = lax.axis_index("s")

        def gather(q, p):
            pltpu.async_copy(ta_h.at[idxr.at[q, 0]], rows2.at[p], gsem)

        def gwait():
            pltpu.make_async_copy(ta_h.at[idxr.at[0, 0]], rows2.at[0],
                                  gsem).wait()

        def iwait():
            pltpu.make_async_copy(ta_h.at[idxr.at[0, 0]], idxr.at[0],
                                  isem).wait()

        def swait():
            pltpu.make_async_copy(rows2.at[0], acc_sh.at[idxr.at[0, 1]],
                                  ssem).wait()

        for h in range(n_halves):
            def ifetch(jj, q):
                pltpu.async_copy(sd_h.at[h, s, c * nch + jj], idxr.at[q], isem)

            pltpu.sync_copy(z_h, acc_sh.at[pl.ds(s * RPT_A, RPT_A)])
            ifetch(0, 0)
            iwait()
            ifetch(jnp.minimum(1, nch - 1), 1)
            plsc.subcore_barrier()
            gather(0, 0)

            def step(j, carry):
                p = lax.rem(j, 2)
                pn = 1 - p
                q = lax.rem(j, 4)
                qn = lax.rem(j + 1, 4)
                iwait()

                @pl.when(j >= 1)
                def _():
                    swait()

                gather(qn, pn)
                gwait()
                pltpu.async_copy(rows2.at[p], acc_sh.at[idxr.at[q, 1]], ssem,
                                 add=True)
                ifetch(jnp.minimum(j + 2, nch - 1), lax.rem(j + 2, 4))
                return carry

            lax.fori_loop(0, nch, step, 0)
            iwait()
            gwait()
            swait()
            plsc.subcore_barrier()
            pltpu.sync_copy(acc_sh.at[pl.ds(s * RPT_A, RPT_A)],
                            out_h.at[h, c, pl.ds(s * RPT_A, RPT_A)])
            if h + 1 < n_halves:
                plsc.subcore_barrier()

    return agg_kernel


def _tc_matmul1(x, W1):
    def body(x_ref, w_ref, h1_ref):
        h1_ref[0] = jnp.dot(x_ref[:], w_ref[:],
                            preferred_element_type=jnp.float32)

    return pl.pallas_call(
        body,
        grid=(2, N_NODES // MB),
        in_specs=[
            pl.BlockSpec((MB, D_IN), lambda h, i: (i, 0)),
            pl.BlockSpec((D_IN, 128), lambda h, i: (0, h)),
        ],
        out_specs=pl.BlockSpec((1, MB, 128), lambda h, i: (h, i, 0)),
        out_shape=jax.ShapeDtypeStruct((2, N_NODES, 128), jnp.float32),
    )(x, W1)


def _tc_prescale(h1, degp):
    def body(h1_ref, degp_ref, hp_ref, dinv_ref):
        deg = degp_ref[0, :, :1] + degp_ref[1, :, :1] + 1.0
        dinv = lax.rsqrt(deg)
        hp_ref[0] = h1_ref[0] * dinv
        dinv_ref[:] = dinv

    return pl.pallas_call(
        body,
        grid=(2, N_NODES // MB),
        in_specs=[
            pl.BlockSpec((1, MB, 128), lambda h, i: (h, i, 0)),
            pl.BlockSpec((2, MB, DEG_W), lambda h, i: (0, i, 0)),
        ],
        out_specs=[
            pl.BlockSpec((1, MB, 128), lambda h, i: (h, i, 0)),
            pl.BlockSpec((MB, 1), lambda h, i: (i, 0)),
        ],
        out_shape=[
            jax.ShapeDtypeStruct((2, N_NODES, 128), jnp.float32),
            jax.ShapeDtypeStruct((N_NODES, 1), jnp.float32),
        ],
    )(h1, degp)


def _tc_bn_relu_matmul(agg1, hp1, dinv, b1s, gs, bes, Wc):
    def body(agg_ref, hp_ref, dinv_ref, b1_ref, g_ref, be_ref, wc_ref, out_ref):
        hid = pl.program_id(0)
        dinv = dinv_ref[:]
        h = (agg_ref[0, 0, :N_NODES, :] + agg_ref[0, 1, :N_NODES, :]
             + hp_ref[0]) * dinv + b1_ref[0, 0]
        mean = jnp.mean(h, axis=0, keepdims=True)
        var = jnp.mean(h * h, axis=0, keepdims=True) - mean * mean
        hn = (h - mean) * lax.rsqrt(var + 1e-5) * g_ref[0, 0] + be_ref[0, 0]
        hr = jnp.maximum(hn, 0.0)
        part = jnp.dot(hr, wc_ref[:], preferred_element_type=jnp.float32)

        @pl.when(hid == 0)
        def _():
            out_ref[:] = part

        @pl.when(hid == 1)
        def _():
            out_ref[:] = (out_ref[:] + part) * dinv

    return pl.pallas_call(
        body,
        grid=(2,),
        in_specs=[
            pl.BlockSpec((1, NC, NACC_A, 128), lambda h: (h, 0, 0, 0)),
            pl.BlockSpec((1, N_NODES, 128), lambda h: (h, 0, 0)),
            pl.BlockSpec((N_NODES, 1), lambda h: (0, 0)),
            pl.BlockSpec((1, 1, 128), lambda h: (h, 0, 0)),
            pl.BlockSpec((1, 1, 128), lambda h: (h, 0, 0)),
            pl.BlockSpec((1, 1, 128), lambda h: (h, 0, 0)),
            pl.BlockSpec((128, 128), lambda h: (h, 0)),
        ],
        out_specs=pl.BlockSpec((N_NODES, 128), lambda h: (0, 0)),
        out_shape=jax.ShapeDtypeStruct((N_NODES, 128), jnp.float32),
    )(agg1, hp1, dinv, b1s, gs, bes, Wc)


def _tc_final(agg2, gp, dinv, bc):
    def body(agg_ref, gp_ref, dinv_ref, b_ref, out_ref):
        v = ((agg_ref[0] + agg_ref[1] + gp_ref[:]) * dinv_ref[:] + b_ref[:])
        out_ref[0] = v[:, :D_LAT]
        out_ref[1] = v[:, D_LAT:]

    return pl.pallas_call(
        body,
        grid=(N_NODES // MB,),
        in_specs=[
            pl.BlockSpec((2, MB, 128), lambda i: (0, i, 0)),
            pl.BlockSpec((MB, 128), lambda i: (i, 0)),
            pl.BlockSpec((MB, 1), lambda i: (i, 0)),
            pl.BlockSpec((1, 128), lambda i: (0, 0)),
        ],
        out_specs=pl.BlockSpec((2, MB, D_LAT), lambda i: (0, i, 0)),
        out_shape=jax.ShapeDtypeStruct((2, N_NODES, D_LAT), jnp.float32),
    )(agg2, gp, dinv, bc)


def kernel(x, edge_index, W1, b1, gamma, beta, W_mu, b_mu, W_ls, b_ls):
    n_edges = edge_index.shape[1]
    e_pad = ((n_edges + NS * CHUNK - 1) // (NS * CHUNK)) * (NS * CHUNK)
    e_pad = ((e_pad + NC * NS * CHUNK - 1) // (NC * NS * CHUNK)) * (NC * NS * CHUNK)
    pad = e_pad - n_edges

    src = edge_index[0]
    dst = edge_index[1]
    ar = jnp.arange(pad, dtype=jnp.int32)
    srcp = jnp.concatenate([src, ar % N_NODES])
    dstp = jnp.concatenate([dst, N_NODES + ar % (NACC_A - N_NODES)])
    src16 = srcp.reshape(NS, -1, CHUNK)
    dst16 = dstp.reshape(NS, -1, CHUNK)
    sd16 = jnp.stack([src16, dst16], axis=2)
    dst32 = dstp.reshape(NC * NS, -1, CHUNK)

    ones_rows = jnp.ones((CHUNK, DEG_W), jnp.float32)
    z16 = jnp.zeros((RPT, DEG_W), jnp.float32)
    z128 = jnp.zeros((RPT_A, 128), jnp.float32)

    h1 = _tc_matmul1(x, W1)
    degp = _deg_hist(dst32, ones_rows, z16)

    hp1, dinv = _tc_prescale(h1, degp)

    sd2 = jnp.stack([sd16, sd16 + jnp.array([N_NODES, 0], jnp.int32)[None, None, :, None]])
    agg1 = _make_agg(src16.shape[1], 2)(hp1.reshape(2 * N_NODES, 128), sd2, z128)

    gp = _tc_bn_relu_matmul(
        agg1, hp1, dinv,
        b1.reshape(2, 1, 128), gamma.reshape(2, 1, 128), beta.reshape(2, 1, 128),
        jnp.concatenate([W_mu, W_ls], axis=1))

    agg2 = _make_agg(src16.shape[1], 1)(gp, sd16[None], z128)
    agg2 = agg2.reshape(NC, NACC_A, 128)

    out = _tc_final(agg2, gp, dinv,
                    jnp.concatenate([b_mu, b_ls]).reshape(1, 128))
    return (out[0], out[1])
